# baseline (speedup 1.0000x reference)
import functools

import numpy as np
import jax
import jax.numpy as jnp
from jax import lax
from jax.experimental import pallas as pl
from jax.experimental.pallas import tpu as pltpu
from jax.experimental.pallas import tpu_sc as plsc

_xs = np.random.RandomState(0).randn(1_000_000)
_silu_np = _xs / (1.0 + np.exp(-_xs))
_SILU_C = float(np.sqrt(np.mean(_silu_np ** 2)))
_INV_SILU_C = 1.0 / _SILU_C

N_NODES = 10000
E = 160000
D = 128
N_EA = 19
CH = 128
NC, NS = 2, 16
NW = NC * NS
E_PAD = 163840
NCHUNK = E_PAD // CH
CPW = NCHUNK // NW
CPT = NCHUNK // NS
ACC_ROWS = 10240
DUMMY_ROW = N_NODES
RPT = N_NODES // NS
ZPT = ACC_ROWS // NS
BE = 1280
BC = BE // CH

_MESH = dict(core_axis_name="c", subcore_axis_name="s",
             num_cores=NC, num_subcores=NS)


GDEPTH = 2


def _gather_body(nf, s2d, out, spt, idx_v, bufs, gsems, wsems):
    cid = lax.axis_index("c")
    sid = lax.axis_index("s")
    w = cid * NS + sid
    base = w * CPW
    @pl.when(sid < NS - 1)
    def _():
        pltpu.sync_copy(nf.at[pl.ds(sid * 640, 640)],
                        spt.at[pl.ds(sid * 640, 640)])

    @pl.when(sid == NS - 1)
    def _():
        pltpu.sync_copy(nf.at[pl.ds((NS - 1) * 640, N_NODES - (NS - 1) * 640)],
                        spt.at[pl.ds((NS - 1) * 640, N_NODES - (NS - 1) * 640)])

    pltpu.sync_copy(s2d.at[pl.ds(base, CPW)], idx_v)
    plsc.subcore_barrier()

    def body(jr, carry):
        j = jr * GDEPTH
        gs = [pltpu.async_copy(spt.at[idx_v.at[j + k]], bufs[k], gsems[k])
              for k in range(GDEPTH)]
        ws = []
        for k in range(GDEPTH):
            gs[k].wait()
            ws.append(pltpu.async_copy(bufs[k], out.at[base + j + k], wsems[k]))
        for wk in ws:
            wk.wait()
        return carry

    lax.fori_loop(0, CPW // GDEPTH, body, 0)


@functools.cache
def _gather():
    return pl.kernel(
        _gather_body,
        out_type=jax.ShapeDtypeStruct((NCHUNK, CH, D), jnp.float32),
        mesh=plsc.VectorSubcoreMesh(**_MESH),
        scratch_types=[
            pltpu.VMEM_SHARED((N_NODES, D), jnp.float32),
            pltpu.VMEM((CPW, CH), jnp.int32),
            [pltpu.VMEM((CH, D), jnp.float32) for _ in range(GDEPTH)],
            [pltpu.SemaphoreType.DMA for _ in range(GDEPTH)],
            [pltpu.SemaphoreType.DMA for _ in range(GDEPTH)],
        ],
    )


def _tc_body(g_ref, ea_ref, w0, w1, w2, w3, out_ref):
    g = g_ref[...].reshape(BE, D)
    ea = ea_ref[...]
    h = jax.nn.silu((ea[:, :16] @ w0[...]) * 0.25) * _INV_SILU_C
    h = jax.nn.silu((h @ w1[...]) * 0.125) * _INV_SILU_C
    h = jax.nn.silu((h @ w2[...]) * 0.125) * _INV_SILU_C
    mix = (h @ w3[...]) * (0.125 * 0.25)
    ms = mix[:, :D]
    mv = mix[:, D:]
    gv = g * mv
    out = jnp.stack(
        [g * ms, gv * ea[:, 16:17], gv * ea[:, 17:18], gv * ea[:, 18:19]]
    )
    out_ref[...] = out.reshape(4, BC, CH, D)


def _tc_call(gathered, ea_p, W0, W1, W2, W3):
    grid = E // BE
    return pl.pallas_call(
        _tc_body,
        grid=(grid,),
        in_specs=[
            pl.BlockSpec((BC, CH, D), lambda i: (i, 0, 0)),
            pl.BlockSpec((BE, N_EA), lambda i: (i, 0)),
            pl.BlockSpec((16, 64), lambda i: (0, 0)),
            pl.BlockSpec((64, 64), lambda i: (0, 0)),
            pl.BlockSpec((64, 64), lambda i: (0, 0)),
            pl.BlockSpec((64, 256), lambda i: (0, 0)),
        ],
        out_specs=pl.BlockSpec((4, BC, CH, D), lambda i: (0, i, 0, 0)),
        out_shape=jax.ShapeDtypeStruct((4, NCHUNK, CH, D), jnp.float32),
    )(gathered, ea_p, W0, W1, W2, W3)


def _scatter_body(msgs, r2d, zeros_hbm, out4, acc, idx_v, bufa, bufb, sla, slb):
    cid = lax.axis_index("c")
    tid = lax.axis_index("s")
    pltpu.sync_copy(r2d.at[pl.ds(tid * CPT, CPT)], idx_v)
    for p in range(2):
        b = cid * 2 + p
        pltpu.sync_copy(zeros_hbm.at[pl.ds(tid * ZPT, ZPT)],
                        acc.at[pl.ds(tid * ZPT, ZPT)])
        plsc.subcore_barrier()

        def body(j2, carry):
            j = j2 * 2
            gid = tid * CPT + j
            la = pltpu.async_copy(msgs.at[b, gid], bufa, sla)
            lb = pltpu.async_copy(msgs.at[b, gid + 1], bufb, slb)
            la.wait()
            pltpu.sync_copy(bufa, acc.at[idx_v.at[j]], add=True)
            lb.wait()
            pltpu.sync_copy(bufb, acc.at[idx_v.at[j + 1]], add=True)
            return carry

        lax.fori_loop(0, CPT // 2, body, 0)
        plsc.subcore_barrier()
        pltpu.sync_copy(acc.at[pl.ds(tid * ZPT, ZPT)],
                        out4.at[b, pl.ds(tid * ZPT, ZPT)])
        plsc.subcore_barrier()


@functools.cache
def _scatter():
    return pl.kernel(
        _scatter_body,
        out_type=jax.ShapeDtypeStruct((4, ACC_ROWS, D), jnp.float32),
        mesh=plsc.VectorSubcoreMesh(**_MESH),
        scratch_types=[
            pltpu.VMEM_SHARED((ACC_ROWS, D), jnp.float32),
            pltpu.VMEM((CPT, CH), jnp.int32),
            pltpu.VMEM((CH, D), jnp.float32),
            pltpu.VMEM((CH, D), jnp.float32),
            pltpu.SemaphoreType.DMA,
            pltpu.SemaphoreType.DMA,
        ],
    )


def _ep_body(o4_ref, p_ref, out_ref):
    s = o4_ref[0]
    v = jnp.concatenate([o4_ref[1], o4_ref[2], o4_ref[3]], axis=1)
    vp = jnp.dot(v, p_ref[...], precision=jax.lax.Precision.HIGHEST)
    out_ref[...] = jnp.concatenate([s, vp], axis=1)


def _ep_call(out4, perm):
    RB = 2000
    return pl.pallas_call(
        _ep_body,
        grid=(N_NODES // RB,),
        in_specs=[
            pl.BlockSpec((4, RB, D), lambda i: (0, i, 0)),
            pl.BlockSpec((3 * D, 3 * D), lambda i: (0, 0)),
        ],
        out_specs=pl.BlockSpec((RB, 4 * D), lambda i: (i, 0)),
        out_shape=jax.ShapeDtypeStruct((N_NODES, 4 * D), jnp.float32),
    )(out4, perm)


_PSRC = np.arange(3 * D)
_PERM_NP = np.zeros((3 * D, 3 * D), dtype=np.float32)
_PERM_NP[(_PSRC % 3) * D + _PSRC // 3, _PSRC] = 1.0


def kernel(node_feats, edge_attrs, senders, receivers, W0, W1, W2, W3):
    pad = E_PAD - E
    s_p = jnp.concatenate(
        [senders, jnp.zeros((pad,), jnp.int32)]).reshape(NCHUNK, CH)
    r_p = jnp.concatenate(
        [receivers, jnp.full((pad,), DUMMY_ROW, jnp.int32)]).reshape(NCHUNK, CH)
    zeros = jnp.zeros((ACC_ROWS, D), jnp.float32)
    gathered = _gather()(node_feats, s_p)
    msgs = _tc_call(gathered, edge_attrs, W0, W1, W2, W3)
    out4 = _scatter()(msgs, r_p, zeros)
    return _ep_call(out4, jnp.asarray(_PERM_NP))


# speedup vs baseline: 1.0801x; 1.0801x over previous
import functools

import numpy as np
import jax
import jax.numpy as jnp
from jax import lax
from jax.experimental import pallas as pl
from jax.experimental.pallas import tpu as pltpu
from jax.experimental.pallas import tpu_sc as plsc

_xs = np.random.RandomState(0).randn(1_000_000)
_silu_np = _xs / (1.0 + np.exp(-_xs))
_SILU_C = float(np.sqrt(np.mean(_silu_np ** 2)))
_INV_SILU_C = 1.0 / _SILU_C

N_NODES = 10000
E = 160000
D = 128
N_EA = 19
CH = 128
NC, NS = 2, 16
NW = NC * NS
E_PAD = 163840
NCHUNK = E_PAD // CH
CPW = NCHUNK // NW
CPT = NCHUNK // NS
ACC_ROWS = 10240
DUMMY_ROW = N_NODES
RPT = N_NODES // NS
ZPT = ACC_ROWS // NS
BE = 1280
BC = BE // CH

_MESH = dict(core_axis_name="c", subcore_axis_name="s",
             num_cores=NC, num_subcores=NS)


GDEPTH = 2


def _gather_body(nf, s2d, out, spt, idx_v, bufs, gsems, wsems):
    cid = lax.axis_index("c")
    sid = lax.axis_index("s")
    w = cid * NS + sid
    base = w * CPW
    @pl.when(sid < NS - 1)
    def _():
        pltpu.sync_copy(nf.at[pl.ds(sid * 640, 640)],
                        spt.at[pl.ds(sid * 640, 640)])

    @pl.when(sid == NS - 1)
    def _():
        pltpu.sync_copy(nf.at[pl.ds((NS - 1) * 640, N_NODES - (NS - 1) * 640)],
                        spt.at[pl.ds((NS - 1) * 640, N_NODES - (NS - 1) * 640)])

    pltpu.sync_copy(s2d.at[pl.ds(base, CPW)], idx_v)
    plsc.subcore_barrier()

    def body(jr, carry):
        j = jr * GDEPTH
        gs = [pltpu.async_copy(spt.at[idx_v.at[j + k]], bufs[k], gsems[k])
              for k in range(GDEPTH)]
        ws = []
        for k in range(GDEPTH):
            gs[k].wait()
            ws.append(pltpu.async_copy(bufs[k], out.at[base + j + k], wsems[k]))
        for wk in ws:
            wk.wait()
        return carry

    lax.fori_loop(0, CPW // GDEPTH, body, 0)


@functools.cache
def _gather():
    return pl.kernel(
        _gather_body,
        out_type=jax.ShapeDtypeStruct((NCHUNK, CH, D), jnp.float32),
        mesh=plsc.VectorSubcoreMesh(**_MESH),
        scratch_types=[
            pltpu.VMEM_SHARED((N_NODES, D), jnp.float32),
            pltpu.VMEM((CPW, CH), jnp.int32),
            [pltpu.VMEM((CH, D), jnp.float32) for _ in range(GDEPTH)],
            [pltpu.SemaphoreType.DMA for _ in range(GDEPTH)],
            [pltpu.SemaphoreType.DMA for _ in range(GDEPTH)],
        ],
    )


def _tc_body(g_ref, ea_ref, w0, w1, w2, w3, out_ref):
    g = g_ref[...].reshape(BE, D)
    ea = ea_ref[...]
    h = jax.nn.silu((ea[:, :16] @ w0[...]) * 0.25) * _INV_SILU_C
    h = jax.nn.silu((h @ w1[...]) * 0.125) * _INV_SILU_C
    h = jax.nn.silu((h @ w2[...]) * 0.125) * _INV_SILU_C
    mix = (h @ w3[...]) * (0.125 * 0.25)
    ms = mix[:, :D]
    mv = mix[:, D:]
    gv = g * mv
    out = jnp.stack(
        [g * ms, gv * ea[:, 16:17], gv * ea[:, 17:18], gv * ea[:, 18:19]]
    )
    out_ref[...] = out.reshape(4, BC, CH, D)


HCHUNK = NCHUNK // 2


def _tc_call(gathered, ea_p, W0, W1, W2, W3, blk_off, nblk):
    return pl.pallas_call(
        _tc_body,
        grid=(nblk,),
        in_specs=[
            pl.BlockSpec((BC, CH, D), lambda i: (i + blk_off, 0, 0)),
            pl.BlockSpec((BE, N_EA), lambda i: (i + blk_off, 0)),
            pl.BlockSpec((16, 64), lambda i: (0, 0)),
            pl.BlockSpec((64, 64), lambda i: (0, 0)),
            pl.BlockSpec((64, 64), lambda i: (0, 0)),
            pl.BlockSpec((64, 256), lambda i: (0, 0)),
        ],
        out_specs=pl.BlockSpec((4, BC, CH, D), lambda i: (0, i, 0, 0)),
        out_shape=jax.ShapeDtypeStruct((4, HCHUNK, CH, D), jnp.float32),
    )(gathered, ea_p, W0, W1, W2, W3)


CPTH = HCHUNK // NS


@functools.cache
def _scatter(chunk_off):
    def body_fn(msgs, r2d, zeros_hbm, out4, acc, idx_v, bufa, bufb, sla, slb):
        cid = lax.axis_index("c")
        tid = lax.axis_index("s")
        pltpu.sync_copy(r2d.at[pl.ds(chunk_off + tid * CPTH, CPTH)], idx_v)
        for p in range(2):
            b = cid * 2 + p
            pltpu.sync_copy(zeros_hbm.at[pl.ds(tid * ZPT, ZPT)],
                            acc.at[pl.ds(tid * ZPT, ZPT)])
            plsc.subcore_barrier()

            def body(j2, carry):
                j = j2 * 2
                gid = tid * CPTH + j
                la = pltpu.async_copy(msgs.at[b, gid], bufa, sla)
                lb = pltpu.async_copy(msgs.at[b, gid + 1], bufb, slb)
                la.wait()
                pltpu.sync_copy(bufa, acc.at[idx_v.at[j]], add=True)
                lb.wait()
                pltpu.sync_copy(bufb, acc.at[idx_v.at[j + 1]], add=True)
                return carry

            lax.fori_loop(0, CPTH // 2, body, 0)
            plsc.subcore_barrier()
            pltpu.sync_copy(acc.at[pl.ds(tid * ZPT, ZPT)],
                            out4.at[b, pl.ds(tid * ZPT, ZPT)])
            plsc.subcore_barrier()

    return pl.kernel(
        body_fn,
        out_type=jax.ShapeDtypeStruct((4, ACC_ROWS, D), jnp.float32),
        mesh=plsc.VectorSubcoreMesh(**_MESH),
        scratch_types=[
            pltpu.VMEM_SHARED((ACC_ROWS, D), jnp.float32),
            pltpu.VMEM((CPTH, CH), jnp.int32),
            pltpu.VMEM((CH, D), jnp.float32),
            pltpu.VMEM((CH, D), jnp.float32),
            pltpu.SemaphoreType.DMA,
            pltpu.SemaphoreType.DMA,
        ],
    )


def _ep_body(oa_ref, ob_ref, p_ref, out_ref):
    o0 = oa_ref[0] + ob_ref[0]
    v = jnp.concatenate(
        [oa_ref[1] + ob_ref[1], oa_ref[2] + ob_ref[2], oa_ref[3] + ob_ref[3]],
        axis=1)
    vp = jnp.dot(v, p_ref[...], precision=jax.lax.Precision.HIGHEST)
    out_ref[...] = jnp.concatenate([o0, vp], axis=1)


def _ep_call(out4a, out4b, perm):
    RB = 2000
    spec4 = pl.BlockSpec((4, RB, D), lambda i: (0, i, 0))
    return pl.pallas_call(
        _ep_body,
        grid=(N_NODES // RB,),
        in_specs=[
            spec4,
            spec4,
            pl.BlockSpec((3 * D, 3 * D), lambda i: (0, 0)),
        ],
        out_specs=pl.BlockSpec((RB, 4 * D), lambda i: (i, 0)),
        out_shape=jax.ShapeDtypeStruct((N_NODES, 4 * D), jnp.float32),
    )(out4a, out4b, perm)


_PSRC = np.arange(3 * D)
_PERM_NP = np.zeros((3 * D, 3 * D), dtype=np.float32)
_PERM_NP[(_PSRC % 3) * D + _PSRC // 3, _PSRC] = 1.0


def kernel(node_feats, edge_attrs, senders, receivers, W0, W1, W2, W3):
    pad = E_PAD - E
    s_p = jnp.concatenate(
        [senders, jnp.zeros((pad,), jnp.int32)]).reshape(NCHUNK, CH)
    r_p = jnp.concatenate(
        [receivers, jnp.full((pad,), DUMMY_ROW, jnp.int32)]).reshape(NCHUNK, CH)
    zeros = jnp.zeros((ACC_ROWS, D), jnp.float32)
    gathered = _gather()(node_feats, s_p)
    nblk0 = HCHUNK // BC
    nblk1 = (E // CH - HCHUNK) // BC
    msgs0 = _tc_call(gathered, edge_attrs, W0, W1, W2, W3, 0, nblk0)
    msgs1 = _tc_call(gathered, edge_attrs, W0, W1, W2, W3, nblk0, nblk1)
    out4a = _scatter(0)(msgs0, r_p, zeros)
    out4b = _scatter(HCHUNK)(msgs1, r_p, zeros)
    return _ep_call(out4a, out4b, jnp.asarray(_PERM_NP))
